# baseline (speedup 1.0000x reference)
import functools

import jax
import jax.numpy as jnp
from jax import lax
from jax.experimental import pallas as pl
from jax.experimental.pallas import tpu as pltpu
from jax.experimental.pallas import tpu_sc as plsc

N_NODES = 10000
FDIM = 128
CHUNK = 128
NC = 2
NS = 16
NW = NC * NS
ROWS_MAIN = (N_NODES // NS) // 8 * 8
ROWS_TAIL = N_NODES - NS * ROWS_MAIN
IB = 4


def _sc_aggregate(row1d, col1d, val1d, features):
    n_edges = row1d.shape[0]
    cpw = n_edges // CHUNK // NW

    mesh = plsc.VectorSubcoreMesh(core_axis_name="c", subcore_axis_name="s")

    @functools.partial(
        pl.kernel,
        mesh=mesh,
        out_type=jax.ShapeDtypeStruct((NC, N_NODES, FDIM), jnp.float32),
        scratch_types=[
            pltpu.VMEM_SHARED((N_NODES, FDIM), jnp.float32),
            pltpu.VMEM((IB, CHUNK), jnp.int32),
            pltpu.VMEM((IB, CHUNK), jnp.int32),
            pltpu.VMEM((IB, CHUNK), jnp.float32),
            pltpu.VMEM((2, CHUNK, FDIM), jnp.float32),
            pltpu.SemaphoreType.DMA,
            pltpu.SemaphoreType.DMA,
            pltpu.SemaphoreType.DMA,
            pltpu.SemaphoreType.DMA,
            pltpu.SemaphoreType.DMA,
            pltpu.SemaphoreType.DMA,
        ],
    )
    def agg(row_hbm, col_hbm, val_hbm, feat_hbm, out_hbm, acc, colv, rowv,
            valv, grows, gsem0, gsem1, isem0, isem1, isem2, isem3):
        cc = lax.axis_index("c")
        sid = lax.axis_index("s")
        wid = sid * NC + cc
        gsems = (gsem0, gsem1)
        isems = (isem0, isem1, isem2, isem3)
        edge_base = wid * cpw * CHUNK

        def zrow(r, _):
            def zcol(i, _):
                grows[0, r, pl.ds(i * 16, 16)] = jnp.zeros((16,), jnp.float32)
                return 0
            return lax.fori_loop(0, FDIM // 16, zcol, 0)
        lax.fori_loop(0, CHUNK, zrow, 0)

        base_row = sid * ROWS_MAIN
        for j in range(ROWS_MAIN // CHUNK):
            pltpu.sync_copy(grows.at[0],
                            acc.at[pl.ds(base_row + j * CHUNK, CHUNK)])
        tail = ROWS_MAIN - (ROWS_MAIN // CHUNK) * CHUNK
        pltpu.sync_copy(
            grows.at[0, pl.ds(0, tail)],
            acc.at[pl.ds(base_row + (ROWS_MAIN // CHUNK) * CHUNK, tail)])

        @pl.when(sid == 0)
        def _():
            pltpu.sync_copy(grows.at[0, pl.ds(0, ROWS_TAIL)],
                            acc.at[pl.ds(NS * ROWS_MAIN, ROWS_TAIL)])
        plsc.subcore_barrier()

        def idx_copies(c, q):
            base = edge_base + c * CHUNK
            return (
                pltpu.make_async_copy(col_hbm.at[pl.ds(base, CHUNK)],
                                      colv.at[q], isems[q]),
                pltpu.make_async_copy(row_hbm.at[pl.ds(base, CHUNK)],
                                      rowv.at[q], isems[q]),
                pltpu.make_async_copy(val_hbm.at[pl.ds(base, CHUNK)],
                                      valv.at[q], isems[q]),
            )

        def idx_start(c, q):
            for cp in idx_copies(c, q):
                cp.start()

        def idx_wait(c, q):
            for cp in idx_copies(c, q):
                cp.wait()

        def gather_copy(q, b):
            return pltpu.make_async_copy(feat_hbm.at[colv.at[q]],
                                         grows.at[b], gsems[b])

        def scale_scatter(q, b):
            def grp_body(grp, _):
                vv = valv[q, pl.ds(grp * 16, 16)]
                for lane in range(16):
                    v = vv[lane]
                    e = grp * 16 + lane
                    for kk in range(FDIM // 16):
                        grows[b, e, pl.ds(kk * 16, 16)] = (
                            grows[b, e, pl.ds(kk * 16, 16)] * v)
                return 0
            lax.fori_loop(0, CHUNK // 16, grp_body, 0)
            pltpu.sync_copy(grows.at[b], acc.at[rowv.at[q]], add=True)

        idx_start(0, 0)
        idx_wait(0, 0)
        idx_start(1, 1)
        gather_copy(0, 0).start()

        def quad_body(i, _):
            for u in range(4):
                c = i * 4 + u
                q = u % IB
                b = u % 2
                @pl.when(c + 2 < cpw)
                def _():
                    idx_start(c + 2, (u + 2) % IB)

                @pl.when(c + 1 < cpw)
                def _():
                    idx_wait(c + 1, (u + 1) % IB)
                    gather_copy((u + 1) % IB, 1 - b).start()
                gather_copy(q, b).wait()
                scale_scatter(q, b)
            return 0

        lax.fori_loop(0, cpw // 4, quad_body, 0)
        plsc.subcore_barrier()

        pltpu.sync_copy(acc.at[pl.ds(base_row, ROWS_MAIN)],
                        out_hbm.at[cc, pl.ds(base_row, ROWS_MAIN)])

        @pl.when(sid == 0)
        def _():
            pltpu.sync_copy(acc.at[pl.ds(NS * ROWS_MAIN, ROWS_TAIL)],
                            out_hbm.at[cc, pl.ds(NS * ROWS_MAIN, ROWS_TAIL)])

    return agg(row1d, col1d, val1d, features)


def _tc_combine_matmul(partials, weight):
    bn = 1000

    def body(p_ref, w_ref, o_ref):
        s = p_ref[0] + p_ref[1]
        o_ref[...] = jnp.maximum(
            jnp.dot(s, w_ref[...], preferred_element_type=jnp.float32), 0.0)

    return pl.pallas_call(
        body,
        grid=(N_NODES // bn,),
        in_specs=[
            pl.BlockSpec((NC, bn, FDIM), lambda i: (0, i, 0)),
            pl.BlockSpec((FDIM, FDIM), lambda i: (0, 0)),
        ],
        out_specs=pl.BlockSpec((bn, FDIM), lambda i: (i, 0)),
        out_shape=jax.ShapeDtypeStruct((N_NODES, FDIM), jnp.float32),
    )(partials, weight)


def kernel(features, adj_indices, adj_values, weight):
    idx = adj_indices.astype(jnp.int32)
    n_edges = idx.shape[1]
    gran = NW * 4 * CHUNK
    n_pad = (n_edges + gran - 1) // gran * gran
    pad = n_pad - n_edges
    row = jnp.pad(idx[0], (0, pad))
    col = jnp.pad(idx[1], (0, pad))
    val = jnp.pad(adj_values, (0, pad))
    partials = _sc_aggregate(row, col, val, features)
    return _tc_combine_matmul(partials, weight)
